# baseline (speedup 1.0000x reference)
import jax
import jax.numpy as jnp
from jax import lax
from jax.experimental import pallas as pl
from jax.experimental.pallas import tpu as pltpu
from jax.experimental.pallas import tpu_sc as plsc

R = 1000000
D = 32
B = 16384

NC = 2
NS = 16
L = 16
NW = NC * NS
BPW = B // NW


def _gmf_body(users_h, items_h, ut_h, it_h, w_h, b_h, out_h,
              uidx, iidx, gu, gi, wv, bv, outv, sem):
    wid = lax.axis_index("s") * NC + lax.axis_index("c")
    base = wid * BPW

    pltpu.sync_copy(users_h.at[pl.ds(base, BPW)], uidx)
    pltpu.sync_copy(items_h.at[pl.ds(base, BPW)], iidx)
    pltpu.sync_copy(w_h, wv)
    pltpu.sync_copy(b_h, bv)

    def fire(k, carry):
        uv = uidx[pl.ds(k * L, L)]
        iv = iidx[pl.ds(k * L, L)]
        for d in range(D):
            pltpu.async_copy(ut_h.at[uv + d * R],
                             gu.at[pl.ds(d * BPW + k * L, L)], sem)
            pltpu.async_copy(it_h.at[iv + d * R],
                             gi.at[pl.ds(d * BPW + k * L, L)], sem)
        return carry

    lax.fori_loop(0, BPW // L, fire, 0)

    w0 = wv[pl.ds(0, L)]
    w1 = wv[pl.ds(L, L)]
    bias = bv[...]

    pltpu.make_async_copy(ut_h.at[pl.ds(0, D * BPW)], gu, sem).wait()
    pltpu.make_async_copy(it_h.at[pl.ds(0, D * BPW)], gi, sem).wait()

    def col(k, carry):
        off = k * L
        acc = bias
        for dd, wreg in ((0, w0), (1, w1)):
            for j in range(L):
                d = dd * L + j
                wd = lax.broadcast(wreg[j], (L,))
                u = gu[pl.ds(d * BPW + off, L)]
                i = gi[pl.ds(d * BPW + off, L)]
                acc = acc + wd * u * i
        o = 1.0 / (1.0 + jnp.exp(-acc))
        outv[pl.ds(off, L)] = o
        return carry

    lax.fori_loop(0, BPW // L, col, 0)

    pltpu.sync_copy(outv, out_h.at[pl.ds(base, BPW)])


@jax.jit
def _gmf(users, items, ut_flat, it_flat, w_flat, b_vec):
    mesh = plsc.VectorSubcoreMesh(core_axis_name="c", subcore_axis_name="s",
                                  num_cores=NC, num_subcores=NS)
    run = pl.kernel(
        _gmf_body,
        out_type=jax.ShapeDtypeStruct((B,), jnp.float32),
        mesh=mesh,
        compiler_params=pltpu.CompilerParams(needs_layout_passes=False),
        scratch_types=[
            pltpu.VMEM((BPW,), jnp.int32),
            pltpu.VMEM((BPW,), jnp.int32),
            pltpu.VMEM((D * BPW,), jnp.float32),
            pltpu.VMEM((D * BPW,), jnp.float32),
            pltpu.VMEM((D,), jnp.float32),
            pltpu.VMEM((L,), jnp.float32),
            pltpu.VMEM((BPW,), jnp.float32),
            pltpu.SemaphoreType.DMA,
        ],
    )
    return run(users, items, ut_flat, it_flat, w_flat, b_vec)


def kernel(items, users, user_table, item_table, W, b):
    ut_flat = user_table.T.reshape(R * D)
    it_flat = item_table.T.reshape(R * D)
    w_flat = W.reshape(D)
    b_vec = jnp.broadcast_to(b.reshape(()), (L,))
    out = _gmf(users.astype(jnp.int32), items.astype(jnp.int32),
               ut_flat, it_flat, w_flat, b_vec)
    return out.reshape(B, 1)


# speedup vs baseline: 5.5937x; 5.5937x over previous
import jax
import jax.numpy as jnp
from jax import lax
from jax.experimental import pallas as pl
from jax.experimental.pallas import tpu as pltpu
from jax.experimental.pallas import tpu_sc as plsc

R = 1000000
D = 32
B = 16384
RPB = 128 // D

NC = 2
NS = 16
L = 16
NW = NC * NS
BPW = B // NW
H = BPW // 2
NKH = H // L


def _gmf_body(users_h, items_h, ut_h, it_h, ws_h, bs_h, out_h,
              uidx, iidx, qu, qi, gu, gi, wsv, bsv, outv, sem):
    wid = lax.axis_index("s") * NC + lax.axis_index("c")
    base = wid * BPW

    pltpu.sync_copy(users_h.at[pl.ds(base, BPW)], uidx)
    pltpu.sync_copy(items_h.at[pl.ds(base, BPW)], iidx)
    pltpu.sync_copy(ws_h, wsv)
    pltpu.sync_copy(bs_h, bsv)

    bias = bsv[...]
    ivec = lax.iota(jnp.int32, L)

    def half(h, carry):
        hb = h * H
        def mkq(k, c):
            jj = pl.ds(hb + k * L, L)
            qu[pl.ds(k * L, L)] = lax.shift_right_logical(uidx[jj], 2)
            qi[pl.ds(k * L, L)] = lax.shift_right_logical(iidx[jj], 2)
            return c

        lax.fori_loop(0, NKH, mkq, 0)

        cu = pltpu.async_copy(ut_h.at[qu], gu, sem)
        ci = pltpu.async_copy(it_h.at[qi], gi, sem)
        cu.wait()
        ci.wait()

        def chunk(k, c):
            jj = pl.ds(hb + k * L, L)
            bvec = ivec + k * L
            uoff = lax.shift_left(uidx[jj] & 3, 5)
            ioff = lax.shift_left(iidx[jj] & 3, 5)
            acc = bias
            for d in range(D):
                u = plsc.load_gather(gu, [bvec, uoff + d])
                i = plsc.load_gather(gi, [bvec, ioff + d])
                acc = acc + wsv[d] * (u * i)
            outv[pl.ds(hb + k * L, L)] = 1.0 / (1.0 + jnp.exp(-acc))
            return c

        lax.fori_loop(0, NKH, chunk, 0)
        return carry

    lax.fori_loop(0, 2, half, 0)

    pltpu.sync_copy(outv, out_h.at[pl.ds(base, BPW)])


@jax.jit
def _gmf(users, items, ut, it, wsplat, bsplat):
    mesh = plsc.VectorSubcoreMesh(core_axis_name="c", subcore_axis_name="s",
                                  num_cores=NC, num_subcores=NS)
    run = pl.kernel(
        _gmf_body,
        out_type=jax.ShapeDtypeStruct((B,), jnp.float32),
        mesh=mesh,
        compiler_params=pltpu.CompilerParams(needs_layout_passes=False),
        scratch_types=[
            pltpu.VMEM((BPW,), jnp.int32),
            pltpu.VMEM((BPW,), jnp.int32),
            pltpu.VMEM((H,), jnp.int32),
            pltpu.VMEM((H,), jnp.int32),
            pltpu.VMEM((H, 128), jnp.float32),
            pltpu.VMEM((H, 128), jnp.float32),
            pltpu.VMEM((D, L), jnp.float32),
            pltpu.VMEM((L,), jnp.float32),
            pltpu.VMEM((BPW,), jnp.float32),
            pltpu.SemaphoreType.DMA,
        ],
    )
    return run(users, items, ut, it, wsplat, bsplat)


def kernel(items, users, user_table, item_table, W, b):
    ut = user_table.reshape(R // RPB, D * RPB)
    it = item_table.reshape(R // RPB, D * RPB)
    wsplat = jnp.broadcast_to(W.reshape(D, 1), (D, L)).astype(jnp.float32)
    bsplat = jnp.broadcast_to(b.reshape(1), (L,)).astype(jnp.float32)
    out = _gmf(users.astype(jnp.int32), items.astype(jnp.int32),
               ut, it, wsplat, bsplat)
    return out.reshape(B, 1)
